# Initial kernel scaffold; baseline (speedup 1.0000x reference)
import jax
import jax.numpy as jnp
from jax.experimental import pallas as pl


def kernel(x, edge_index, edge_weight, W1, b1, W2, b2, W3, b3, Wh, bh):
    raise NotImplementedError("write your pallas kernel here")


# speedup vs baseline: 12.6605x; 12.6605x over previous
import functools

import jax
import jax.numpy as jnp
from jax import lax
from jax.experimental import pallas as pl
from jax.experimental.pallas import tpu as pltpu
from jax.experimental.pallas import tpu_sc as plsc

N = 10000
NP = 10240
C = 128
CHUNK = 128
EPT = 10368
NCHUNK = EPT // CHUNK
NPHASE = 3
PCHUNK = NCHUNK // NPHASE
EP = 32 * EPT
RPT = NP // 16
BLK = 1024

_mesh = plsc.VectorSubcoreMesh(core_axis_name="c", subcore_axis_name="s")


@functools.partial(
    pl.kernel,
    out_type=jax.ShapeDtypeStruct((2, NP, C), jnp.float32),
    mesh=_mesh,
    scratch_types=[
        pltpu.VMEM_SHARED((NP, C), jnp.float32),
        pltpu.VMEM((PCHUNK, CHUNK), jnp.int32),
        pltpu.VMEM((PCHUNK, CHUNK), jnp.float32),
        pltpu.VMEM((CHUNK, C), jnp.float32),
    ],
)
def _deg_kernel(dst_hbm, ew_hbm, zeros_hbm, out_hbm, deg_sh, dstb, ewb, wbuf):
    c = lax.axis_index("c")
    s = lax.axis_index("s")
    w = c * 16 + s

    rbase = pl.multiple_of(s * RPT, 8)
    pltpu.sync_copy(zeros_hbm.at[pl.ds(rbase, RPT)], deg_sh.at[pl.ds(rbase, RPT)])
    plsc.subcore_barrier()

    def phase_body(p, _):
        pltpu.sync_copy(dst_hbm.at[w, p], dstb)
        pltpu.sync_copy(ew_hbm.at[w, p], ewb)

        def chunk_body(k, _):
            for g in range(CHUNK // 16):
                wv = ewb[k, pl.ds(g * 16, 16)]
                for l in range(16):
                    row = jnp.full((16,), wv[l], jnp.float32)
                    for j in range(C // 16):
                        wbuf[g * 16 + l, pl.ds(j * 16, 16)] = row
            pltpu.sync_copy(wbuf, deg_sh.at[dstb.at[k]], add=True)
            return 0

        lax.fori_loop(0, PCHUNK, chunk_body, 0)
        return 0

    lax.fori_loop(0, NPHASE, phase_body, 0)
    plsc.subcore_barrier()
    pltpu.sync_copy(deg_sh.at[pl.ds(rbase, RPT)], out_hbm.at[c, pl.ds(rbase, RPT)])


@functools.partial(
    pl.kernel,
    out_type=jax.ShapeDtypeStruct((2, NP, C), jnp.float32),
    mesh=_mesh,
    scratch_types=[
        pltpu.VMEM_SHARED((NP, C), jnp.float32),
        pltpu.VMEM((PCHUNK, CHUNK), jnp.int32),
        pltpu.VMEM((PCHUNK, CHUNK), jnp.int32),
        pltpu.VMEM((PCHUNK, CHUNK), jnp.float32),
        pltpu.VMEM((CHUNK, C), jnp.float32),
        pltpu.SemaphoreType.DMA,
    ],
)
def _msg_kernel(h_hbm, src_hbm, dst_hbm, ew_hbm, zeros_hbm, out_hbm,
                acc, srcb, dstb, ewb, gbuf, sem):
    c = lax.axis_index("c")
    s = lax.axis_index("s")
    w = c * 16 + s

    rbase = pl.multiple_of(s * RPT, 8)
    pltpu.sync_copy(zeros_hbm.at[pl.ds(rbase, RPT)], acc.at[pl.ds(rbase, RPT)])
    plsc.subcore_barrier()

    def phase_body(p, _):
        pltpu.sync_copy(src_hbm.at[w, p], srcb)
        pltpu.sync_copy(dst_hbm.at[w, p], dstb)
        pltpu.sync_copy(ew_hbm.at[w, p], ewb)

        def chunk_body(k, _):
            pltpu.async_copy(h_hbm.at[srcb.at[k]], gbuf, sem).wait()
            for g in range(CHUNK // 16):
                wv = ewb[k, pl.ds(g * 16, 16)]
                for l in range(16):
                    row = g * 16 + l
                    wgt = wv[l]
                    for j in range(C // 16):
                        gbuf[row, pl.ds(j * 16, 16)] = gbuf[row, pl.ds(j * 16, 16)] * wgt
            pltpu.sync_copy(gbuf, acc.at[dstb.at[k]], add=True)
            return 0

        lax.fori_loop(0, PCHUNK, chunk_body, 0)
        return 0

    lax.fori_loop(0, NPHASE, phase_body, 0)
    plsc.subcore_barrier()
    pltpu.sync_copy(acc.at[pl.ds(rbase, RPT)], out_hbm.at[c, pl.ds(rbase, RPT)])


def _dinv_from_parts(degp):
    deg = jnp.sum(degp[:, :, :1], axis=0)
    return jnp.where(deg > 0, lax.rsqrt(deg), 0.0)


def _tc1_body(degp_ref, x_ref, w1_ref, out_ref):
    dinv = _dinv_from_parts(degp_ref[...])
    h = jnp.dot(x_ref[...], w1_ref[...], preferred_element_type=jnp.float32)
    out_ref[...] = dinv * h


def _tc2_body(accp_ref, degp_ref, b1_ref, w2_ref, out_ref):
    dinv = _dinv_from_parts(degp_ref[...])
    a = jnp.sum(accp_ref[...], axis=0)
    h2 = jnp.maximum(dinv * a + b1_ref[...], 0.0)
    out_ref[...] = dinv * jnp.dot(h2, w2_ref[...], preferred_element_type=jnp.float32)


def _tc3_body(accp_ref, degp_ref, b2_ref, w3_ref, b3_ref, wh_ref, bh_ref, out_ref):
    dinv = _dinv_from_parts(degp_ref[...])
    a = jnp.sum(accp_ref[...], axis=0)
    h3 = dinv * a + b2_ref[...]
    h4 = jnp.dot(h3, w3_ref[...], preferred_element_type=jnp.float32) + b3_ref[...]
    lg = jnp.dot(h4, wh_ref[...], preferred_element_type=jnp.float32) + bh_ref[...]
    m = jnp.max(lg, axis=1, keepdims=True)
    e = jnp.exp(lg - m)
    out_ref[...] = e / jnp.sum(e, axis=1, keepdims=True)


_GRID = (NP // BLK,)


def _full(shape):
    return pl.BlockSpec(shape, lambda i: (0,) * len(shape))


def _rows(shape):
    return pl.BlockSpec(shape, lambda i: (i,) + (0,) * (len(shape) - 1))


def _rows1(shape):
    return pl.BlockSpec(shape, lambda i: (0, i, 0))


_tc1 = pl.pallas_call(
    _tc1_body,
    grid=_GRID,
    in_specs=[_rows1((2, BLK, C)), _rows((BLK, C)), _full((C, C))],
    out_specs=_rows((BLK, C)),
    out_shape=jax.ShapeDtypeStruct((NP, C), jnp.float32),
)

_tc2 = pl.pallas_call(
    _tc2_body,
    grid=_GRID,
    in_specs=[_rows1((2, BLK, C)), _rows1((2, BLK, C)), _full((1, C)),
              _full((C, C))],
    out_specs=_rows((BLK, C)),
    out_shape=jax.ShapeDtypeStruct((NP, C), jnp.float32),
)

_tc3 = pl.pallas_call(
    _tc3_body,
    grid=_GRID,
    in_specs=[_rows1((2, BLK, C)), _rows1((2, BLK, C)), _full((1, C)),
              _full((C, C // 2)), _full((1, C // 2)), _full((C // 2, 10)),
              _full((1, 10))],
    out_specs=_rows((BLK, 10)),
    out_shape=jax.ShapeDtypeStruct((NP, 10), jnp.float32),
)


def kernel(x, edge_index, edge_weight, W1, b1, W2, b2, W3, b3, Wh, bh):
    n = x.shape[0]
    loop = jnp.arange(n, dtype=jnp.int32)
    src = jnp.concatenate([edge_index[0].astype(jnp.int32), loop])
    dst = jnp.concatenate([edge_index[1].astype(jnp.int32), loop])
    ew = jnp.concatenate([edge_weight, jnp.ones((n,), edge_weight.dtype)])

    pad_e = EP - src.shape[0]
    src = jnp.pad(src, (0, pad_e)).reshape(32, NPHASE, PCHUNK, CHUNK)
    dst = jnp.pad(dst, (0, pad_e)).reshape(32, NPHASE, PCHUNK, CHUNK)
    ew = jnp.pad(ew, (0, pad_e)).reshape(32, NPHASE, PCHUNK, CHUNK)
    xp = jnp.pad(x, ((0, NP - n), (0, 0)))
    znc = jnp.zeros((NP, C), jnp.float32)

    degp = _deg_kernel(dst, ew, znc)
    h1 = _tc1(degp, xp, W1)
    acc1 = _msg_kernel(h1, src, dst, ew, znc)
    h2 = _tc2(acc1, degp, b1.reshape(1, C), W2)
    acc2 = _msg_kernel(h2, src, dst, ew, znc)
    out = _tc3(acc2, degp, b2.reshape(1, C), W3,
               b3.reshape(1, C // 2), Wh, bh.reshape(1, 10))
    return out[:n]
